# baseline (speedup 1.0000x reference)
import functools
import numpy as np
import jax
import jax.numpy as jnp
from jax.experimental import pallas as pl
from jax.experimental.pallas import tpu as pltpu

_BN_EPS = 1e-5


def _interp_mat(out_size: int, in_size: int) -> np.ndarray:
    i = np.arange(out_size, dtype=np.float64)
    scale = in_size / out_size
    src = np.maximum((i + 0.5) * scale - 0.5, 0.0)
    i0 = np.minimum(np.floor(src).astype(np.int64), in_size - 1)
    i1 = np.minimum(i0 + 1, in_size - 1)
    lam = src - i0
    M = np.zeros((out_size, in_size), dtype=np.float32)
    rows = np.arange(out_size)
    np.add.at(M, (rows, i0), (1.0 - lam).astype(np.float32))
    np.add.at(M, (rows, i1), lam.astype(np.float32))
    return M


def _resize_mat(Hout, Wout, Hin, Win) -> np.ndarray:
    wh = _interp_mat(Hout, Hin)
    ww = _interp_mat(Wout, Win)
    return np.kron(wh, ww).T.astype(np.float32)


def _tap_masks(Hout, Wout) -> np.ndarray:
    HW = Hout * Wout
    h = np.repeat(np.arange(Hout), Wout)
    w = np.tile(np.arange(Wout), Hout)
    masks = np.zeros((9, HW), np.float32)
    t = 0
    for dy in (-1, 0, 1):
        for dx in (-1, 0, 1):
            hv = (h + dy >= 0) & (h + dy < Hout)
            wv = (w + dx >= 0) & (w + dx < Wout)
            masks[t] = (hv & wv).astype(np.float32)
            t += 1
    return masks


def _fwd_kernel(x_ref, r_ref, w_ref, m_ref, pre_ref, stat_ref,
                *, Wout, C, P):
    HW = pre_ref.shape[-1]

    resized = jnp.dot(x_ref[...], r_ref[...],
                      preferred_element_type=jnp.float32)
    resized = resized.astype(jnp.bfloat16)
    masks = m_ref[...]

    s1 = jnp.zeros((C, 1), jnp.float32)
    s2 = jnp.zeros((C, 1), jnp.float32)
    for p in range(P):
        rz = resized[p * C:(p + 1) * C]
        taps = []
        t = 0
        for dy in (-1, 0, 1):
            for dx in (-1, 0, 1):
                d = dy * Wout + dx
                if d == 0:
                    taps.append(rz)
                else:
                    shifted = pltpu.roll(rz, shift=(-d) % HW, axis=1)
                    taps.append(shifted * masks[t:t + 1, :])
                t += 1
        cols = jnp.concatenate(taps, axis=0)

        pre = jnp.dot(w_ref[...], cols,
                      preferred_element_type=jnp.float32)
        pre_ref[p * C:(p + 1) * C, :] = pre.astype(jnp.bfloat16)

        s1 = s1 + jnp.sum(pre, axis=1, keepdims=True)
        s2 = s2 + jnp.sum(pre * pre, axis=1, keepdims=True)
    stat_ref[...] = jnp.concatenate([s1, s2], axis=1)


def _bn_relu_kernel(pre_ref, stat_ref, g_ref, b_ref, o_ref, *, inv_count, P):
    st = jnp.sum(stat_ref[...], axis=0)
    mean = st[:, 0:1] * inv_count
    var = st[:, 1:2] * inv_count - mean * mean
    scale = jax.lax.rsqrt(var + _BN_EPS) * g_ref[...]
    shift = b_ref[...] - mean * scale
    scale_p = jnp.concatenate([scale] * P, axis=0)
    shift_p = jnp.concatenate([shift] * P, axis=0)
    pre = pre_ref[...].astype(jnp.float32)
    o_ref[...] = jnp.maximum(pre * scale_p + shift_p, 0.0)


def _upblock(x_nchw, out_size, params):
    N, Cin, Hin, Win = x_nchw.shape
    Hout, Wout = out_size
    HW = Hout * Wout
    w = params['w']
    Cout = w.shape[0]
    assert Cin == Cout, "this kernel assumes Cin == Cout"
    C = Cout
    P = 2
    NP = N // P
    Kin = Hin * Win

    x3 = x_nchw.reshape(NP, P * Cin, Kin).astype(jnp.bfloat16)

    r = jnp.asarray(_resize_mat(Hout, Wout, Hin, Win)).astype(jnp.bfloat16)
    masks = jnp.asarray(_tap_masks(Hout, Wout)).astype(jnp.bfloat16)

    w2 = jnp.transpose(w, (0, 2, 3, 1)).reshape(Cout, 9 * Cin)
    w2 = w2.astype(jnp.bfloat16)

    gamma = params['gamma'].reshape(Cout, 1).astype(jnp.float32)
    beta = params['beta'].reshape(Cout, 1).astype(jnp.float32)

    cparams = pltpu.CompilerParams(
        dimension_semantics=("parallel",),
        vmem_limit_bytes=64 * 1024 * 1024,
    )

    pre, stats = pl.pallas_call(
        functools.partial(_fwd_kernel, Wout=Wout, C=C, P=P),
        grid=(NP,),
        in_specs=[
            pl.BlockSpec((None, P * Cin, Kin), lambda n: (n, 0, 0)),
            pl.BlockSpec((Kin, HW), lambda n: (0, 0)),
            pl.BlockSpec((Cout, 9 * Cin), lambda n: (0, 0)),
            pl.BlockSpec((9, HW), lambda n: (0, 0)),
        ],
        out_specs=(
            pl.BlockSpec((None, P * Cout, HW), lambda n: (n, 0, 0)),
            pl.BlockSpec((None, Cout, 2), lambda n: (n, 0, 0)),
        ),
        out_shape=(
            jax.ShapeDtypeStruct((NP, P * Cout, HW), jnp.bfloat16),
            jax.ShapeDtypeStruct((NP, Cout, 2), jnp.float32),
        ),
        compiler_params=cparams,
    )(x3, r, w2, masks)

    inv_count = 1.0 / float(N * HW)
    out = pl.pallas_call(
        functools.partial(_bn_relu_kernel, inv_count=inv_count, P=P),
        grid=(NP,),
        in_specs=[
            pl.BlockSpec((None, P * Cout, HW), lambda n: (n, 0, 0)),
            pl.BlockSpec((NP, Cout, 2), lambda n: (0, 0, 0)),
            pl.BlockSpec((Cout, 1), lambda n: (0, 0)),
            pl.BlockSpec((Cout, 1), lambda n: (0, 0)),
        ],
        out_specs=pl.BlockSpec((None, P * Cout, HW), lambda n: (n, 0, 0)),
        out_shape=jax.ShapeDtypeStruct((NP, P * Cout, HW), jnp.float32),
        compiler_params=cparams,
    )(pre, stats, gamma, beta)

    return out.reshape(N, Cout, Hout, Wout)


def kernel(x, w, b, gamma, beta):
    params = {"w": w, "b": b, "gamma": gamma, "beta": beta}
    return _upblock(x, (64, 64), params)


# speedup vs baseline: 1.6088x; 1.6088x over previous
import functools
import numpy as np
import jax
import jax.numpy as jnp
from jax.experimental import pallas as pl
from jax.experimental.pallas import tpu as pltpu

_BN_EPS = 1e-5


def _interp_mat(out_size: int, in_size: int) -> np.ndarray:
    i = np.arange(out_size, dtype=np.float64)
    scale = in_size / out_size
    src = np.maximum((i + 0.5) * scale - 0.5, 0.0)
    i0 = np.minimum(np.floor(src).astype(np.int64), in_size - 1)
    i1 = np.minimum(i0 + 1, in_size - 1)
    lam = src - i0
    M = np.zeros((out_size, in_size), dtype=np.float32)
    rows = np.arange(out_size)
    np.add.at(M, (rows, i0), (1.0 - lam).astype(np.float32))
    np.add.at(M, (rows, i1), lam.astype(np.float32))
    return M


def _resize_mat(Hout, Wout, Hin, Win) -> np.ndarray:
    wh = _interp_mat(Hout, Hin)
    ww = _interp_mat(Wout, Win)
    return np.kron(wh, ww).T.astype(np.float32)


def _tap_masks(Hout, Wout) -> np.ndarray:
    HW = Hout * Wout
    h = np.repeat(np.arange(Hout), Wout)
    w = np.tile(np.arange(Wout), Hout)
    masks = np.zeros((9, HW), np.float32)
    t = 0
    for dy in (-1, 0, 1):
        for dx in (-1, 0, 1):
            hv = (h + dy >= 0) & (h + dy < Hout)
            wv = (w + dx >= 0) & (w + dx < Wout)
            masks[t] = (hv & wv).astype(np.float32)
            t += 1
    return masks


def _fwd_kernel(x_ref, r_ref, w_ref, m_ref, pre_ref, stat_ref,
                *, Wout, C, P):
    HW = pre_ref.shape[-1]
    Kin = x_ref.shape[-1]

    xb = x_ref[...].reshape(P * C, Kin).astype(jnp.bfloat16)
    resized = jnp.dot(xb, r_ref[...],
                      preferred_element_type=jnp.float32)
    resized = resized.astype(jnp.bfloat16)
    masks = m_ref[...]

    s1 = jnp.zeros((C, 1), jnp.float32)
    s2 = jnp.zeros((C, 1), jnp.float32)
    for p in range(P):
        rz = resized[p * C:(p + 1) * C]
        taps = []
        t = 0
        for dy in (-1, 0, 1):
            for dx in (-1, 0, 1):
                d = dy * Wout + dx
                if d == 0:
                    taps.append(rz)
                else:
                    shifted = pltpu.roll(rz, shift=(-d) % HW, axis=1)
                    taps.append(shifted * masks[t:t + 1, :])
                t += 1
        cols = jnp.concatenate(taps, axis=0)

        pre = jnp.dot(w_ref[...], cols,
                      preferred_element_type=jnp.float32)
        pre_ref[p] = pre.astype(jnp.bfloat16)

        s1 = s1 + jnp.sum(pre, axis=1, keepdims=True)
        s2 = s2 + jnp.sum(pre * pre, axis=1, keepdims=True)
    stat_ref[...] = jnp.concatenate([s1, s2], axis=1)


def _bn_relu_kernel(pre_ref, stat_ref, g_ref, b_ref, o_ref, *, inv_count, P):
    st = jnp.sum(stat_ref[...], axis=0)
    mean = st[:, 0:1] * inv_count
    var = st[:, 1:2] * inv_count - mean * mean
    scale = jax.lax.rsqrt(var + _BN_EPS) * g_ref[...]
    shift = b_ref[...] - mean * scale
    for p in range(P):
        pre = pre_ref[p].astype(jnp.float32)
        o_ref[p] = jnp.maximum(pre * scale + shift, 0.0)


def _upblock(x_nchw, out_size, params):
    N, Cin, Hin, Win = x_nchw.shape
    Hout, Wout = out_size
    HW = Hout * Wout
    w = params['w']
    Cout = w.shape[0]
    assert Cin == Cout, "this kernel assumes Cin == Cout"
    C = Cout
    P = 2
    NP = N // P
    Kin = Hin * Win

    x3 = x_nchw.reshape(N, Cin, Kin)

    r = jnp.asarray(_resize_mat(Hout, Wout, Hin, Win)).astype(jnp.bfloat16)
    masks = jnp.asarray(_tap_masks(Hout, Wout)).astype(jnp.bfloat16)

    w2 = jnp.transpose(w, (0, 2, 3, 1)).reshape(Cout, 9 * Cin)
    w2 = w2.astype(jnp.bfloat16)

    gamma = params['gamma'].reshape(Cout, 1).astype(jnp.float32)
    beta = params['beta'].reshape(Cout, 1).astype(jnp.float32)

    cparams = pltpu.CompilerParams(
        dimension_semantics=("parallel",),
        vmem_limit_bytes=64 * 1024 * 1024,
    )

    pre, stats = pl.pallas_call(
        functools.partial(_fwd_kernel, Wout=Wout, C=C, P=P),
        grid=(NP,),
        in_specs=[
            pl.BlockSpec((P, Cin, Kin), lambda n: (n, 0, 0)),
            pl.BlockSpec((Kin, HW), lambda n: (0, 0)),
            pl.BlockSpec((Cout, 9 * Cin), lambda n: (0, 0)),
            pl.BlockSpec((9, HW), lambda n: (0, 0)),
        ],
        out_specs=(
            pl.BlockSpec((P, Cout, HW), lambda n: (n, 0, 0)),
            pl.BlockSpec((None, Cout, 2), lambda n: (n, 0, 0)),
        ),
        out_shape=(
            jax.ShapeDtypeStruct((N, Cout, HW), jnp.bfloat16),
            jax.ShapeDtypeStruct((NP, Cout, 2), jnp.float32),
        ),
        compiler_params=cparams,
    )(x3, r, w2, masks)

    inv_count = 1.0 / float(N * HW)
    out = pl.pallas_call(
        functools.partial(_bn_relu_kernel, inv_count=inv_count, P=P),
        grid=(NP,),
        in_specs=[
            pl.BlockSpec((P, Cout, HW), lambda n: (n, 0, 0)),
            pl.BlockSpec((NP, Cout, 2), lambda n: (0, 0, 0)),
            pl.BlockSpec((Cout, 1), lambda n: (0, 0)),
            pl.BlockSpec((Cout, 1), lambda n: (0, 0)),
        ],
        out_specs=pl.BlockSpec((P, Cout, HW), lambda n: (n, 0, 0)),
        out_shape=jax.ShapeDtypeStruct((N, Cout, HW), jnp.float32),
        compiler_params=cparams,
    )(pre, stats, gamma, beta)

    return out.reshape(N, Cout, Hout, Wout)


def kernel(x, w, b, gamma, beta):
    params = {"w": w, "b": b, "gamma": gamma, "beta": beta}
    return _upblock(x, (64, 64), params)


# speedup vs baseline: 1.8212x; 1.1321x over previous
import functools
import numpy as np
import jax
import jax.numpy as jnp
from jax.experimental import pallas as pl
from jax.experimental.pallas import tpu as pltpu

_BN_EPS = 1e-5


def _interp_mat(out_size: int, in_size: int) -> np.ndarray:
    i = np.arange(out_size, dtype=np.float64)
    scale = in_size / out_size
    src = np.maximum((i + 0.5) * scale - 0.5, 0.0)
    i0 = np.minimum(np.floor(src).astype(np.int64), in_size - 1)
    i1 = np.minimum(i0 + 1, in_size - 1)
    lam = src - i0
    M = np.zeros((out_size, in_size), dtype=np.float32)
    rows = np.arange(out_size)
    np.add.at(M, (rows, i0), (1.0 - lam).astype(np.float32))
    np.add.at(M, (rows, i1), lam.astype(np.float32))
    return M


def _resize_mat(Hout, Wout, Hin, Win) -> np.ndarray:
    wh = _interp_mat(Hout, Hin)
    ww = _interp_mat(Wout, Win)
    return np.kron(wh, ww).T.astype(np.float32)


def _banded_resize(Hout, Wout, Hin, Win, G):
    R = _resize_mat(Hout, Wout, Hin, Win)
    Kin = Hin * Win
    HWg = Hout * Wout // G
    rows_per_g = Hout // G
    in_rows = rows_per_g // 2 + 2
    Kb = in_rows * Win
    starts, slabs = [], []
    for g in range(G):
        s = min(max(0, (rows_per_g * g // 2 - 1) * Win), Kin - Kb)
        slab = R[:, HWg * g: HWg * (g + 1)]
        nz = np.nonzero(np.any(slab != 0.0, axis=1))[0]
        assert nz.min() >= s and nz.max() < s + Kb, (g, s, nz.min(), nz.max())
        starts.append(s)
        slabs.append(slab[s:s + Kb])
    return starts, np.stack(slabs)


def _tap_masks(Hout, Wout) -> np.ndarray:
    HW = Hout * Wout
    h = np.repeat(np.arange(Hout), Wout)
    w = np.tile(np.arange(Wout), Hout)
    masks = np.zeros((9, HW), np.float32)
    t = 0
    for dy in (-1, 0, 1):
        for dx in (-1, 0, 1):
            hv = (h + dy >= 0) & (h + dy < Hout)
            wv = (w + dx >= 0) & (w + dx < Wout)
            masks[t] = (hv & wv).astype(np.float32)
            t += 1
    return masks


def _fwd_kernel(x_ref, r_ref, w_ref, m_ref, pre_ref, stat_ref,
                *, Wout, C, P, starts):
    HW = pre_ref.shape[-1]
    Kin = x_ref.shape[-1]
    Kb = r_ref.shape[1]

    xb = x_ref[...].reshape(P * C, Kin).astype(jnp.bfloat16)
    pieces = [
        jnp.dot(xb[:, s:s + Kb], r_ref[g],
                preferred_element_type=jnp.float32)
        for g, s in enumerate(starts)
    ]
    resized = jnp.concatenate(pieces, axis=1)
    resized = resized.astype(jnp.bfloat16)
    masks = m_ref[...]

    s1 = jnp.zeros((C, 1), jnp.float32)
    s2 = jnp.zeros((C, 1), jnp.float32)
    for p in range(P):
        rz = resized[p * C:(p + 1) * C]
        taps = []
        t = 0
        for dy in (-1, 0, 1):
            for dx in (-1, 0, 1):
                d = dy * Wout + dx
                if d == 0:
                    taps.append(rz)
                else:
                    shifted = pltpu.roll(rz, shift=(-d) % HW, axis=1)
                    taps.append(shifted * masks[t:t + 1, :])
                t += 1
        cols = jnp.concatenate(taps, axis=0)

        pre = jnp.dot(w_ref[...], cols,
                      preferred_element_type=jnp.float32)
        pre_ref[p] = pre.astype(jnp.bfloat16)

        s1 = s1 + jnp.sum(pre, axis=1, keepdims=True)
        s2 = s2 + jnp.sum(pre * pre, axis=1, keepdims=True)
    stat_ref[...] = jnp.concatenate([s1, s2], axis=1)


def _bn_relu_kernel(pre_ref, stat_ref, g_ref, b_ref, o_ref, *, inv_count, P):
    st = jnp.sum(stat_ref[...], axis=0)
    mean = st[:, 0:1] * inv_count
    var = st[:, 1:2] * inv_count - mean * mean
    scale = jax.lax.rsqrt(var + _BN_EPS) * g_ref[...]
    shift = b_ref[...] - mean * scale
    for p in range(P):
        pre = pre_ref[p].astype(jnp.float32)
        o_ref[p] = jnp.maximum(pre * scale + shift, 0.0)


def _upblock(x_nchw, out_size, params):
    N, Cin, Hin, Win = x_nchw.shape
    Hout, Wout = out_size
    HW = Hout * Wout
    w = params['w']
    Cout = w.shape[0]
    assert Cin == Cout, "this kernel assumes Cin == Cout"
    C = Cout
    P = 2
    NP = N // P
    Kin = Hin * Win

    x3 = x_nchw.reshape(N, Cin, Kin)

    G = 8
    starts, slabs = _banded_resize(Hout, Wout, Hin, Win, G)
    r = jnp.asarray(slabs).astype(jnp.bfloat16)
    Kb, HWg = r.shape[1], r.shape[2]
    masks = jnp.asarray(_tap_masks(Hout, Wout)).astype(jnp.bfloat16)

    w2 = jnp.transpose(w, (0, 2, 3, 1)).reshape(Cout, 9 * Cin)
    w2 = w2.astype(jnp.bfloat16)

    gamma = params['gamma'].reshape(Cout, 1).astype(jnp.float32)
    beta = params['beta'].reshape(Cout, 1).astype(jnp.float32)

    cparams = pltpu.CompilerParams(
        dimension_semantics=("parallel",),
        vmem_limit_bytes=64 * 1024 * 1024,
    )

    pre, stats = pl.pallas_call(
        functools.partial(_fwd_kernel, Wout=Wout, C=C, P=P,
                          starts=tuple(starts)),
        grid=(NP,),
        in_specs=[
            pl.BlockSpec((P, Cin, Kin), lambda n: (n, 0, 0)),
            pl.BlockSpec((G, Kb, HWg), lambda n: (0, 0, 0)),
            pl.BlockSpec((Cout, 9 * Cin), lambda n: (0, 0)),
            pl.BlockSpec((9, HW), lambda n: (0, 0)),
        ],
        out_specs=(
            pl.BlockSpec((P, Cout, HW), lambda n: (n, 0, 0)),
            pl.BlockSpec((None, Cout, 2), lambda n: (n, 0, 0)),
        ),
        out_shape=(
            jax.ShapeDtypeStruct((N, Cout, HW), jnp.bfloat16),
            jax.ShapeDtypeStruct((NP, Cout, 2), jnp.float32),
        ),
        compiler_params=cparams,
    )(x3, r, w2, masks)

    inv_count = 1.0 / float(N * HW)
    out = pl.pallas_call(
        functools.partial(_bn_relu_kernel, inv_count=inv_count, P=P),
        grid=(NP,),
        in_specs=[
            pl.BlockSpec((P, Cout, HW), lambda n: (n, 0, 0)),
            pl.BlockSpec((NP, Cout, 2), lambda n: (0, 0, 0)),
            pl.BlockSpec((Cout, 1), lambda n: (0, 0)),
            pl.BlockSpec((Cout, 1), lambda n: (0, 0)),
        ],
        out_specs=pl.BlockSpec((P, Cout, HW), lambda n: (n, 0, 0)),
        out_shape=jax.ShapeDtypeStruct((N, Cout, HW), jnp.float32),
        compiler_params=cparams,
    )(pre, stats, gamma, beta)

    return out.reshape(N, Cout, Hout, Wout)


def kernel(x, w, b, gamma, beta):
    params = {"w": w, "b": b, "gamma": gamma, "beta": beta}
    return _upblock(x, (64, 64), params)


# speedup vs baseline: 2.0417x; 1.1210x over previous
import functools
import numpy as np
import jax
import jax.numpy as jnp
from jax.experimental import pallas as pl
from jax.experimental.pallas import tpu as pltpu

_BN_EPS = 1e-5


def _interp_mat(out_size: int, in_size: int) -> np.ndarray:
    i = np.arange(out_size, dtype=np.float64)
    scale = in_size / out_size
    src = np.maximum((i + 0.5) * scale - 0.5, 0.0)
    i0 = np.minimum(np.floor(src).astype(np.int64), in_size - 1)
    i1 = np.minimum(i0 + 1, in_size - 1)
    lam = src - i0
    M = np.zeros((out_size, in_size), dtype=np.float32)
    rows = np.arange(out_size)
    np.add.at(M, (rows, i0), (1.0 - lam).astype(np.float32))
    np.add.at(M, (rows, i1), lam.astype(np.float32))
    return M


def _resize_mat(Hout, Wout, Hin, Win) -> np.ndarray:
    wh = _interp_mat(Hout, Hin)
    ww = _interp_mat(Wout, Win)
    return np.kron(wh, ww).T.astype(np.float32)


def _banded_resize(Hout, Wout, Hin, Win, G):
    R = _resize_mat(Hout, Wout, Hin, Win)
    Kin = Hin * Win
    HWg = Hout * Wout // G
    rows_per_g = Hout // G
    in_rows = rows_per_g // 2 + 2
    Kb = in_rows * Win
    starts, slabs = [], []
    for g in range(G):
        s = min(max(0, (rows_per_g * g // 2 - 1) * Win), Kin - Kb)
        slab = R[:, HWg * g: HWg * (g + 1)]
        nz = np.nonzero(np.any(slab != 0.0, axis=1))[0]
        assert nz.min() >= s and nz.max() < s + Kb, (g, s, nz.min(), nz.max())
        starts.append(s)
        slabs.append(slab[s:s + Kb])
    return starts, np.stack(slabs)


def _tap_masks(Hout, Wout) -> np.ndarray:
    HW = Hout * Wout
    h = np.repeat(np.arange(Hout), Wout)
    w = np.tile(np.arange(Wout), Hout)
    masks = np.zeros((9, HW), np.float32)
    t = 0
    for dy in (-1, 0, 1):
        for dx in (-1, 0, 1):
            hv = (h + dy >= 0) & (h + dy < Hout)
            wv = (w + dx >= 0) & (w + dx < Wout)
            masks[t] = (hv & wv).astype(np.float32)
            t += 1
    return masks


def _fwd_kernel(x_ref, r_ref, w_ref, m_ref, pre_ref, stat_ref,
                *, Wout, C, P, starts):
    HW = pre_ref.shape[-1]
    Kin = x_ref.shape[-1]
    Kb = r_ref.shape[1]

    xb = x_ref[...].reshape(P * C, Kin).astype(jnp.bfloat16)
    pieces = [
        jnp.dot(xb[:, s:s + Kb], r_ref[g],
                preferred_element_type=jnp.float32)
        for g, s in enumerate(starts)
    ]
    resized = jnp.concatenate(pieces, axis=1)
    resized = resized.astype(jnp.bfloat16)
    masks = m_ref[...]

    s1 = jnp.zeros((C, 1), jnp.float32)
    s2 = jnp.zeros((C, 1), jnp.float32)
    for p in range(P):
        rz = resized[p * C:(p + 1) * C]
        taps = []
        t = 0
        for dy in (-1, 0, 1):
            for dx in (-1, 0, 1):
                d = dy * Wout + dx
                if d == 0:
                    taps.append(rz)
                else:
                    shifted = pltpu.roll(rz, shift=(-d) % HW, axis=1)
                    taps.append(shifted * masks[t:t + 1, :])
                t += 1
        cols = jnp.concatenate(taps, axis=0)

        pre = jnp.dot(w_ref[...], cols,
                      preferred_element_type=jnp.float32)
        pre_ref[p] = pre.astype(jnp.bfloat16)

        s1 = s1 + jnp.sum(pre, axis=1, keepdims=True)
        s2 = s2 + jnp.sum(pre * pre, axis=1, keepdims=True)
    stat_ref[...] = jnp.concatenate([s1, s2], axis=1)


def _bn_relu_kernel(pre_ref, stat_ref, g_ref, b_ref, o_ref, *, inv_count, P):
    st = jnp.sum(stat_ref[...], axis=0)
    mean = st[:, 0:1] * inv_count
    var = st[:, 1:2] * inv_count - mean * mean
    scale = jax.lax.rsqrt(var + _BN_EPS) * g_ref[...]
    shift = b_ref[...] - mean * scale
    for p in range(P):
        pre = pre_ref[p].astype(jnp.float32)
        o_ref[p] = jnp.maximum(pre * scale + shift, 0.0)


def _upblock(x_nchw, out_size, params):
    N, Cin, Hin, Win = x_nchw.shape
    Hout, Wout = out_size
    HW = Hout * Wout
    w = params['w']
    Cout = w.shape[0]
    assert Cin == Cout, "this kernel assumes Cin == Cout"
    C = Cout
    P = 4
    NP = N // P
    Kin = Hin * Win

    x3 = x_nchw.reshape(N, Cin, Kin)

    G = 8
    starts, slabs = _banded_resize(Hout, Wout, Hin, Win, G)
    r = jnp.asarray(slabs).astype(jnp.bfloat16)
    Kb, HWg = r.shape[1], r.shape[2]
    masks = jnp.asarray(_tap_masks(Hout, Wout)).astype(jnp.bfloat16)

    w2 = jnp.transpose(w, (0, 2, 3, 1)).reshape(Cout, 9 * Cin)
    w2 = w2.astype(jnp.bfloat16)

    gamma = params['gamma'].reshape(Cout, 1).astype(jnp.float32)
    beta = params['beta'].reshape(Cout, 1).astype(jnp.float32)

    cparams = pltpu.CompilerParams(
        dimension_semantics=("parallel",),
        vmem_limit_bytes=64 * 1024 * 1024,
    )

    pre, stats = pl.pallas_call(
        functools.partial(_fwd_kernel, Wout=Wout, C=C, P=P,
                          starts=tuple(starts)),
        grid=(NP,),
        in_specs=[
            pl.BlockSpec((P, Cin, Kin), lambda n: (n, 0, 0)),
            pl.BlockSpec((G, Kb, HWg), lambda n: (0, 0, 0)),
            pl.BlockSpec((Cout, 9 * Cin), lambda n: (0, 0)),
            pl.BlockSpec((9, HW), lambda n: (0, 0)),
        ],
        out_specs=(
            pl.BlockSpec((P, Cout, HW), lambda n: (n, 0, 0)),
            pl.BlockSpec((None, Cout, 2), lambda n: (n, 0, 0)),
        ),
        out_shape=(
            jax.ShapeDtypeStruct((N, Cout, HW), jnp.bfloat16),
            jax.ShapeDtypeStruct((NP, Cout, 2), jnp.float32),
        ),
        compiler_params=cparams,
    )(x3, r, w2, masks)

    inv_count = 1.0 / float(N * HW)
    out = pl.pallas_call(
        functools.partial(_bn_relu_kernel, inv_count=inv_count, P=P),
        grid=(NP,),
        in_specs=[
            pl.BlockSpec((P, Cout, HW), lambda n: (n, 0, 0)),
            pl.BlockSpec((NP, Cout, 2), lambda n: (0, 0, 0)),
            pl.BlockSpec((Cout, 1), lambda n: (0, 0)),
            pl.BlockSpec((Cout, 1), lambda n: (0, 0)),
        ],
        out_specs=pl.BlockSpec((P, Cout, HW), lambda n: (n, 0, 0)),
        out_shape=jax.ShapeDtypeStruct((N, Cout, HW), jnp.float32),
        compiler_params=cparams,
    )(pre, stats, gamma, beta)

    return out.reshape(N, Cout, Hout, Wout)


def kernel(x, w, b, gamma, beta):
    params = {"w": w, "b": b, "gamma": gamma, "beta": beta}
    return _upblock(x, (64, 64), params)


# speedup vs baseline: 2.1111x; 1.0340x over previous
import functools
import numpy as np
import jax
import jax.numpy as jnp
from jax.experimental import pallas as pl
from jax.experimental.pallas import tpu as pltpu

_BN_EPS = 1e-5


def _interp_mat(out_size: int, in_size: int) -> np.ndarray:
    i = np.arange(out_size, dtype=np.float64)
    scale = in_size / out_size
    src = np.maximum((i + 0.5) * scale - 0.5, 0.0)
    i0 = np.minimum(np.floor(src).astype(np.int64), in_size - 1)
    i1 = np.minimum(i0 + 1, in_size - 1)
    lam = src - i0
    M = np.zeros((out_size, in_size), dtype=np.float32)
    rows = np.arange(out_size)
    np.add.at(M, (rows, i0), (1.0 - lam).astype(np.float32))
    np.add.at(M, (rows, i1), lam.astype(np.float32))
    return M


def _resize_mat(Hout, Wout, Hin, Win) -> np.ndarray:
    wh = _interp_mat(Hout, Hin)
    ww = _interp_mat(Wout, Win)
    return np.kron(wh, ww).T.astype(np.float32)


def _banded_resize(Hout, Wout, Hin, Win, G):
    R = _resize_mat(Hout, Wout, Hin, Win)
    Kin = Hin * Win
    HWg = Hout * Wout // G
    rows_per_g = Hout // G
    in_rows = rows_per_g // 2 + 2
    Kb = in_rows * Win
    starts, slabs = [], []
    for g in range(G):
        s = min(max(0, (rows_per_g * g // 2 - 1) * Win), Kin - Kb)
        slab = R[:, HWg * g: HWg * (g + 1)]
        nz = np.nonzero(np.any(slab != 0.0, axis=1))[0]
        assert nz.min() >= s and nz.max() < s + Kb, (g, s, nz.min(), nz.max())
        starts.append(s)
        slabs.append(slab[s:s + Kb])
    return starts, np.stack(slabs)


def _tap_masks(Hout, Wout) -> np.ndarray:
    HW = Hout * Wout
    h = np.repeat(np.arange(Hout), Wout)
    w = np.tile(np.arange(Wout), Hout)
    masks = np.zeros((9, HW), np.float32)
    t = 0
    for dy in (-1, 0, 1):
        for dx in (-1, 0, 1):
            hv = (h + dy >= 0) & (h + dy < Hout)
            wv = (w + dx >= 0) & (w + dx < Wout)
            masks[t] = (hv & wv).astype(np.float32)
            t += 1
    return masks


def _fwd_kernel(x_ref, r_ref, w_ref, m_ref, pre_ref, stat_ref,
                *, Wout, C, P, starts):
    HW = pre_ref.shape[-1]
    Kin = x_ref.shape[-1]
    Kb = r_ref.shape[1]

    xb = x_ref[...].reshape(P * C, Kin).astype(jnp.bfloat16)
    pieces = [
        jnp.dot(xb[:, s:s + Kb], r_ref[g],
                preferred_element_type=jnp.float32)
        for g, s in enumerate(starts)
    ]
    resized = jnp.concatenate(pieces, axis=1)
    resized = resized.astype(jnp.bfloat16)
    masks = m_ref[...]

    s1 = jnp.zeros((C, 1), jnp.float32)
    s2 = jnp.zeros((C, 1), jnp.float32)
    for p in range(P):
        rz = resized[p * C:(p + 1) * C]
        taps = []
        t = 0
        for dy in (-1, 0, 1):
            for dx in (-1, 0, 1):
                d = dy * Wout + dx
                if d == 0:
                    taps.append(rz)
                else:
                    shifted = pltpu.roll(rz, shift=(-d) % HW, axis=1)
                    taps.append(shifted * masks[t:t + 1, :])
                t += 1
        cols = jnp.concatenate(taps, axis=0)

        pre = jnp.dot(w_ref[...], cols,
                      preferred_element_type=jnp.float32)
        pre_ref[p] = pre.astype(jnp.bfloat16)

        s1 = s1 + jnp.sum(pre, axis=1, keepdims=True)
        s2 = s2 + jnp.sum(pre * pre, axis=1, keepdims=True)
    stat_ref[...] = jnp.concatenate([s1, s2], axis=1)


def _bn_relu_kernel(pre_ref, stat_ref, g_ref, b_ref, o_ref, *, inv_count, P):
    st = jnp.sum(stat_ref[...], axis=0)
    mean = st[:, 0:1] * inv_count
    var = st[:, 1:2] * inv_count - mean * mean
    scale = jax.lax.rsqrt(var + _BN_EPS) * g_ref[...]
    shift = b_ref[...] - mean * scale
    for p in range(P):
        pre = pre_ref[p].astype(jnp.float32)
        o_ref[p] = jnp.maximum(pre * scale + shift, 0.0)


def _upblock(x_nchw, out_size, params):
    N, Cin, Hin, Win = x_nchw.shape
    Hout, Wout = out_size
    HW = Hout * Wout
    w = params['w']
    Cout = w.shape[0]
    assert Cin == Cout, "this kernel assumes Cin == Cout"
    C = Cout
    P = 8
    NP = N // P
    Kin = Hin * Win

    x3 = x_nchw.reshape(N, Cin, Kin)

    G = 8
    starts, slabs = _banded_resize(Hout, Wout, Hin, Win, G)
    r = jnp.asarray(slabs).astype(jnp.bfloat16)
    Kb, HWg = r.shape[1], r.shape[2]
    masks = jnp.asarray(_tap_masks(Hout, Wout)).astype(jnp.bfloat16)

    w2 = jnp.transpose(w, (0, 2, 3, 1)).reshape(Cout, 9 * Cin)
    w2 = w2.astype(jnp.bfloat16)

    gamma = params['gamma'].reshape(Cout, 1).astype(jnp.float32)
    beta = params['beta'].reshape(Cout, 1).astype(jnp.float32)

    cparams = pltpu.CompilerParams(
        dimension_semantics=("parallel",),
        vmem_limit_bytes=64 * 1024 * 1024,
    )

    pre, stats = pl.pallas_call(
        functools.partial(_fwd_kernel, Wout=Wout, C=C, P=P,
                          starts=tuple(starts)),
        grid=(NP,),
        in_specs=[
            pl.BlockSpec((P, Cin, Kin), lambda n: (n, 0, 0)),
            pl.BlockSpec((G, Kb, HWg), lambda n: (0, 0, 0)),
            pl.BlockSpec((Cout, 9 * Cin), lambda n: (0, 0)),
            pl.BlockSpec((9, HW), lambda n: (0, 0)),
        ],
        out_specs=(
            pl.BlockSpec((P, Cout, HW), lambda n: (n, 0, 0)),
            pl.BlockSpec((None, Cout, 2), lambda n: (n, 0, 0)),
        ),
        out_shape=(
            jax.ShapeDtypeStruct((N, Cout, HW), jnp.bfloat16),
            jax.ShapeDtypeStruct((NP, Cout, 2), jnp.float32),
        ),
        compiler_params=cparams,
    )(x3, r, w2, masks)

    inv_count = 1.0 / float(N * HW)
    out = pl.pallas_call(
        functools.partial(_bn_relu_kernel, inv_count=inv_count, P=P),
        grid=(NP,),
        in_specs=[
            pl.BlockSpec((P, Cout, HW), lambda n: (n, 0, 0)),
            pl.BlockSpec((NP, Cout, 2), lambda n: (0, 0, 0)),
            pl.BlockSpec((Cout, 1), lambda n: (0, 0)),
            pl.BlockSpec((Cout, 1), lambda n: (0, 0)),
        ],
        out_specs=pl.BlockSpec((P, Cout, HW), lambda n: (n, 0, 0)),
        out_shape=jax.ShapeDtypeStruct((N, Cout, HW), jnp.float32),
        compiler_params=cparams,
    )(pre, stats, gamma, beta)

    return out.reshape(N, Cout, Hout, Wout)


def kernel(x, w, b, gamma, beta):
    params = {"w": w, "b": b, "gamma": gamma, "beta": beta}
    return _upblock(x, (64, 64), params)
